# Initial kernel scaffold; baseline (speedup 1.0000x reference)
import jax
import jax.numpy as jnp
from jax.experimental import pallas as pl


def kernel(x, edge_index, W1, a_src, a_dst, b1, W2, b2, Wf1, bf1, Wf2, bf2):
    raise NotImplementedError("write your pallas kernel here")


# speedup vs baseline: 44.1999x; 44.1999x over previous
import functools

import jax
import jax.numpy as jnp
from jax import lax
from jax.experimental import pallas as pl
from jax.experimental.pallas import tpu as pltpu
from jax.experimental.pallas import tpu_sc as plsc

NC = 2
NS = 16
LANES = 16

_SC_PARAMS = pltpu.CompilerParams(needs_layout_passes=False,
                                  use_tc_tiling_on_sc=False)


def _leaky(x):
    return jnp.where(x > 0, x, 0.2 * x)


def _row_chunks(total, batch):
    out, off = [], 0
    while off < total:
        step = min(batch, total - off)
        out.append((off, step))
        off += step
    return out


def _zero_fill(zsrc, acc, s, ch, tail, n, batch):
    for off, step in _row_chunks(ch, batch):
        pltpu.sync_copy(zsrc.at[pl.ds(0, step)],
                        acc.at[pl.ds(s * ch + off, step)])
    if tail:
        @pl.when(s == 0)
        def _():
            pltpu.sync_copy(zsrc.at[pl.ds(0, tail)],
                            acc.at[pl.ds(NS * ch, tail)])


def _read_out(acc, dst_at, s, ch, tail, batch):
    for off, step in _row_chunks(ch, batch):
        pltpu.sync_copy(acc.at[pl.ds(s * ch + off, step)],
                        dst_at(s * ch + off, step))
    if tail:
        @pl.when(s == 0)
        def _():
            pltpu.sync_copy(acc.at[pl.ds(NS * ch, tail)],
                            dst_at(NS * ch, tail))


def _tc1_body(x_b, w1_b, asm_b, adm_b, hcat_b, sas_b, sad_b):
    h = jnp.dot(x_b[...], w1_b[...], preferred_element_type=jnp.float32)
    hcat_b[0] = h[:, :128]
    hcat_b[1] = h[:, 128:]
    sas_b[...] = jnp.dot(h, asm_b[...], preferred_element_type=jnp.float32)
    sad_b[...] = jnp.dot(h, adm_b[...], preferred_element_type=jnp.float32)


def _tc1(x, W1, As, Ad, bn):
    n, d_in = x.shape
    d_out = W1.shape[1]
    grid = n // bn
    return pl.pallas_call(
        _tc1_body,
        grid=(grid,),
        in_specs=[
            pl.BlockSpec((bn, d_in), lambda i: (i, 0)),
            pl.BlockSpec((d_in, d_out), lambda i: (0, 0)),
            pl.BlockSpec((d_out, 16), lambda i: (0, 0)),
            pl.BlockSpec((d_out, 16), lambda i: (0, 0)),
        ],
        out_specs=[
            pl.BlockSpec((2, bn, 128), lambda i: (0, i, 0)),
            pl.BlockSpec((bn, 16), lambda i: (i, 0)),
            pl.BlockSpec((bn, 16), lambda i: (i, 0)),
        ],
        out_shape=[
            jax.ShapeDtypeStruct((2, n, 128), jnp.float32),
            jax.ShapeDtypeStruct((n, 16), jnp.float32),
            jax.ShapeDtypeStruct((n, 16), jnp.float32),
        ],
    )(x, W1, As, Ad)


def _sc_a(src, dst, sas, sad, n, batch):
    e = src.shape[0]
    epw = e // (NC * NS)
    nb = epw // batch
    ch = (n // (8 * NS)) * 8
    tail = n - ch * NS
    mesh = plsc.VectorSubcoreMesh(core_axis_name="c", subcore_axis_name="s")

    @functools.partial(
        pl.kernel,
        mesh=mesh,
        compiler_params=_SC_PARAMS,
        out_type=(
            jax.ShapeDtypeStruct((e, 16), jnp.float32),
            jax.ShapeDtypeStruct((NC, n, 16), jnp.float32),
        ),
        scratch_types=[
            pltpu.VMEM((batch,), jnp.int32),
            pltpu.VMEM((batch,), jnp.int32),
            pltpu.VMEM((batch, 16), jnp.float32),
            pltpu.VMEM((batch, 16), jnp.float32),
            pltpu.VMEM((batch, 16), jnp.float32),
            pltpu.VMEM_SHARED((n, 16), jnp.float32),
            pltpu.SemaphoreType.DMA,
            pltpu.SemaphoreType.DMA,
        ],
    )
    def body(src_h, dst_h, sas_h, sad_h, ee_h, esum_h,
             srcb, dstb, asb, adb, sbuf, acc, sem1, sem2):
        c = lax.axis_index("c")
        s = lax.axis_index("s")
        wid = c * NS + s
        lane = lax.broadcasted_iota(jnp.int32, (LANES,), 0)

        def zrow(b, carry):
            sbuf[b] = jnp.zeros((LANES,), jnp.float32)
            return carry
        lax.fori_loop(0, batch, zrow, 0)
        _zero_fill(sbuf, acc, s, ch, tail, n, batch)
        plsc.subcore_barrier()

        def batch_body(i, carry):
            base = wid * epw + i * batch
            pltpu.sync_copy(src_h.at[pl.ds(base, batch)], srcb)
            pltpu.sync_copy(dst_h.at[pl.ds(base, batch)], dstb)
            cp1 = pltpu.async_copy(sas_h.at[srcb], asb, sem1)
            cp2 = pltpu.async_copy(sad_h.at[dstb], adb, sem2)
            cp1.wait()
            cp2.wait()

            def edge(b, carry2):
                ee = jnp.exp(_leaky(asb[b] + adb[b]))
                row = jnp.where(lane < 4, ee,
                                jnp.where(lane == 4, 1.0, 0.0))
                sbuf[b] = row
                return carry2
            lax.fori_loop(0, batch, edge, 0)

            pltpu.sync_copy(sbuf, acc.at[dstb], add=True)
            pltpu.sync_copy(sbuf, ee_h.at[pl.ds(base, batch)])
            return carry
        lax.fori_loop(0, nb, batch_body, 0)

        plsc.subcore_barrier()
        _read_out(acc, lambda o, st: esum_h.at[c, pl.ds(o, st)],
                  s, ch, tail, batch)

    return body(src, dst, sas, sad)


def _sc_b(src2, dst, ee, hcat2, n, batch):
    e = dst.shape[0]
    ept = e // NS
    nb = ept // batch
    ch = (n // (8 * NS)) * 8
    tail = n - ch * NS
    mesh = plsc.VectorSubcoreMesh(core_axis_name="c", subcore_axis_name="s")

    @functools.partial(
        pl.kernel,
        mesh=mesh,
        compiler_params=_SC_PARAMS,
        out_type=jax.ShapeDtypeStruct((NC, n, 128), jnp.float32),
        scratch_types=[
            pltpu.VMEM((batch,), jnp.int32),
            pltpu.VMEM((batch,), jnp.int32),
            pltpu.VMEM((batch, 16), jnp.float32),
            pltpu.VMEM((batch, 128), jnp.float32),
            pltpu.VMEM_SHARED((n, 128), jnp.float32),
            pltpu.SemaphoreType.DMA,
        ],
    )
    def body(src2_h, dst_h, ee_h, hcat2_h, msg_h,
             srcb, dstb, eeb, rows, acc, sem):
        c = lax.axis_index("c")
        s = lax.axis_index("s")

        def zrow(b, carry):
            for k in range(8):
                rows[b, pl.ds(k * LANES, LANES)] = jnp.zeros((LANES,),
                                                             jnp.float32)
            return carry
        lax.fori_loop(0, batch, zrow, 0)
        _zero_fill(rows, acc, s, ch, tail, n, batch)
        plsc.subcore_barrier()

        def batch_body(i, carry):
            base = s * ept + i * batch
            pltpu.sync_copy(src2_h.at[pl.ds(c * e + base, batch)], srcb)
            pltpu.sync_copy(dst_h.at[pl.ds(base, batch)], dstb)
            pltpu.sync_copy(ee_h.at[pl.ds(base, batch)], eeb)
            pltpu.async_copy(hcat2_h.at[srcb], rows, sem).wait()

            def edge(b, carry2):
                for hh in range(2):
                    col = 2 * c + hh
                    av = plsc.load_gather(
                        eeb, [jnp.broadcast_to(b, (LANES,)),
                              jnp.broadcast_to(col, (LANES,))])
                    for k in range(4):
                        sl = pl.ds(hh * 64 + k * LANES, LANES)
                        rows[b, sl] = rows[b, sl] * av
                return carry2
            lax.fori_loop(0, batch, edge, 0)

            pltpu.sync_copy(rows, acc.at[dstb], add=True)
            return carry
        lax.fori_loop(0, nb, batch_body, 0)

        plsc.subcore_barrier()
        _read_out(acc, lambda o, st: msg_h.at[c, pl.ds(o, st)],
                  s, ch, tail, batch)

    return body(src2, dst, ee, hcat2)


def _tc2_body(hcat_b, msg_b, esum_b, sas_b, sad_b, w2_b, b1_b,
              g2_b, dinv8_b):
    es = esum_b[0] + esum_b[1]
    eel = jnp.exp(_leaky(sas_b[...] + sad_b[...]))
    f = es + eel
    inv = 1.0 / (f + 1e-16)
    dinv = lax.rsqrt(f[:, 4:5])
    parts = []
    for hd in range(4):
        cc, hh = hd // 2, hd % 2
        msg_h = msg_b[cc][:, hh * 64:(hh + 1) * 64]
        h_h = hcat_b[cc][:, hh * 64:(hh + 1) * 64]
        parts.append((msg_h + h_h * eel[:, hd:hd + 1]) * inv[:, hd:hd + 1])
    a1 = jnp.maximum(jnp.concatenate(parts, axis=1) + b1_b[...], 0.0)
    h2 = jnp.dot(a1, w2_b[...], preferred_element_type=jnp.float32)
    g2_b[...] = h2 * dinv
    dinv8_b[...] = jnp.broadcast_to(dinv, (dinv.shape[0], 8))


def _tc2(hcat, msg, esum2, sas, sad, W2, b1, bn):
    n = sas.shape[0]
    grid = n // bn
    return pl.pallas_call(
        _tc2_body,
        grid=(grid,),
        in_specs=[
            pl.BlockSpec((2, bn, 128), lambda i: (0, i, 0)),
            pl.BlockSpec((2, bn, 128), lambda i: (0, i, 0)),
            pl.BlockSpec((2, bn, 16), lambda i: (0, i, 0)),
            pl.BlockSpec((bn, 16), lambda i: (i, 0)),
            pl.BlockSpec((bn, 16), lambda i: (i, 0)),
            pl.BlockSpec((256, 64), lambda i: (0, 0)),
            pl.BlockSpec((1, 256), lambda i: (0, 0)),
        ],
        out_specs=[
            pl.BlockSpec((bn, 64), lambda i: (i, 0)),
            pl.BlockSpec((bn, 8), lambda i: (i, 0)),
        ],
        out_shape=[
            jax.ShapeDtypeStruct((n, 64), jnp.float32),
            jax.ShapeDtypeStruct((n, 8), jnp.float32),
        ],
    )(hcat, msg, esum2, sas, sad, W2, b1)


def _sc_c(src, dst, g2, n, batch):
    e = src.shape[0]
    epw = e // (NC * NS)
    nb = epw // batch
    ch = (n // (8 * NS)) * 8
    tail = n - ch * NS
    mesh = plsc.VectorSubcoreMesh(core_axis_name="c", subcore_axis_name="s")

    @functools.partial(
        pl.kernel,
        mesh=mesh,
        compiler_params=_SC_PARAMS,
        out_type=jax.ShapeDtypeStruct((NC, n, 64), jnp.float32),
        scratch_types=[
            pltpu.VMEM((batch,), jnp.int32),
            pltpu.VMEM((batch,), jnp.int32),
            pltpu.VMEM((batch, 64), jnp.float32),
            pltpu.VMEM_SHARED((n, 64), jnp.float32),
            pltpu.SemaphoreType.DMA,
        ],
    )
    def body(src_h, dst_h, g2_h, out_h, srcb, dstb, rows, acc, sem):
        c = lax.axis_index("c")
        s = lax.axis_index("s")
        wid = c * NS + s

        def zrow(b, carry):
            for k in range(4):
                rows[b, pl.ds(k * LANES, LANES)] = jnp.zeros((LANES,),
                                                             jnp.float32)
            return carry
        lax.fori_loop(0, batch, zrow, 0)
        _zero_fill(rows, acc, s, ch, tail, n, batch)
        plsc.subcore_barrier()

        def batch_body(i, carry):
            base = wid * epw + i * batch
            pltpu.sync_copy(src_h.at[pl.ds(base, batch)], srcb)
            pltpu.sync_copy(dst_h.at[pl.ds(base, batch)], dstb)
            pltpu.async_copy(g2_h.at[srcb], rows, sem).wait()
            pltpu.sync_copy(rows, acc.at[dstb], add=True)
            return carry
        lax.fori_loop(0, nb, batch_body, 0)

        plsc.subcore_barrier()
        _read_out(acc, lambda o, st: out_h.at[c, pl.ds(o, st)],
                  s, ch, tail, batch)

    return body(src, dst, g2)


def _tc3_body(p_b, g2_b, dinv8_b, b2_b, wf1_b, bf1_b, wf2_b, bf2_b, out_b):
    dinv = dinv8_b[:, 0:1]
    t = jnp.maximum(dinv * (p_b[0] + p_b[1] + g2_b[...]) + b2_b[...], 0.0)
    u = jnp.maximum(
        jnp.dot(t, wf1_b[...], preferred_element_type=jnp.float32)
        + bf1_b[...], 0.0)
    v = jnp.dot(u, wf2_b[...], preferred_element_type=jnp.float32) + bf2_b[...]
    out_b[...] = jax.nn.sigmoid(v)


def _tc3(p, g2, dinv8, b2, Wf1, bf1, Wf2, bf2, bn):
    n = g2.shape[0]
    grid = n // bn
    return pl.pallas_call(
        _tc3_body,
        grid=(grid,),
        in_specs=[
            pl.BlockSpec((2, bn, 64), lambda i: (0, i, 0)),
            pl.BlockSpec((bn, 64), lambda i: (i, 0)),
            pl.BlockSpec((bn, 8), lambda i: (i, 0)),
            pl.BlockSpec((1, 64), lambda i: (0, 0)),
            pl.BlockSpec((64, 32), lambda i: (0, 0)),
            pl.BlockSpec((1, 32), lambda i: (0, 0)),
            pl.BlockSpec((32, 1), lambda i: (0, 0)),
            pl.BlockSpec((1, 1), lambda i: (0, 0)),
        ],
        out_specs=pl.BlockSpec((bn, 1), lambda i: (i, 0)),
        out_shape=jax.ShapeDtypeStruct((n, 1), jnp.float32),
    )(p, g2, dinv8, b2, Wf1, bf1, Wf2, bf2)


def kernel(x, edge_index, W1, a_src, a_dst, b1, W2, b2, Wf1, bf1, Wf2, bf2):
    n = x.shape[0]
    e = edge_index.shape[1]
    heads, hid = a_src.shape

    src = edge_index[0]
    dst = edge_index[1]
    src2 = jnp.concatenate([src, src + n])

    eye = jnp.eye(heads, 16, dtype=jnp.float32)
    As = (a_src[:, :, None] * eye[:, None, :]).reshape(heads * hid, 16)
    Ad = (a_dst[:, :, None] * eye[:, None, :]).reshape(heads * hid, 16)

    bn = 400
    hcat, sas, sad = _tc1(x, W1, As, Ad, bn)

    ee, esum2 = _sc_a(src, dst, sas, sad, n, 400)

    hcat2 = hcat.reshape(2 * n, 128)
    msg = _sc_b(src2, dst, ee, hcat2, n, 200)

    g2, dinv8 = _tc2(hcat, msg, esum2, sas, sad, W2,
                     b1.reshape(1, heads * hid), bn)

    p = _sc_c(src, dst, g2, n, 400)

    out = _tc3(p, g2, dinv8, b2.reshape(1, hid), Wf1, bf1.reshape(1, 32),
               Wf2, bf2.reshape(1, 1), bn)
    return out
